# baseline (speedup 1.0000x reference)
import functools

import jax
import jax.numpy as jnp
from jax import lax
from jax.experimental import pallas as pl
from jax.experimental.pallas import tpu as pltpu
from jax.experimental.pallas import tpu_sc as plsc

N_EMB = 8192
DIM = 32
N_TOK = 8192
TOK_BLK = 1024
CHUNK = 2048
N_CHUNKS = N_EMB // CHUNK
N_BLKS = N_TOK // TOK_BLK

SC_CORES = 2
SC_SUBCORES = 16
SC_WORKERS = SC_CORES * SC_SUBCORES
ROWS_PER_WORKER = N_TOK // SC_WORKERS


def _tc_body(x_ref, xsq_ref, esq_ref, emb_ref, idx_ref, loss_ref):
    i = pl.program_id(0)
    x = x_ref[...]
    xsq = xsq_ref[...]

    def step(c, carry):
        av, ai = carry
        emb_c = emb_ref[pl.ds(c * CHUNK, CHUNK), :]
        esq_c = esq_ref[:, pl.ds(c * CHUNK, CHUNK)]
        xe = lax.dot_general(x, emb_c, (((1,), (1,)), ((), ())),
                             preferred_element_type=jnp.float32)
        d2 = (xsq - 2.0 * xe) + esq_c
        dist = jnp.sqrt(jnp.maximum(d2, 0.0))
        v = jnp.min(dist, axis=1, keepdims=True)
        cols = lax.broadcasted_iota(jnp.int32, (TOK_BLK, CHUNK), 1)
        g = jnp.min(jnp.where(dist == v, cols, N_EMB), axis=1,
                    keepdims=True) + c * CHUNK
        better = (v < av) | ((v == av) & (g < ai))
        av = jnp.where(better, v, av)
        ai = jnp.where(better, g, ai)
        av = av.astype(jnp.bfloat16).astype(jnp.float32)
        return av, ai

    init = (jnp.full((TOK_BLK, 1), jnp.inf, jnp.float32),
            jnp.zeros((TOK_BLK, 1), jnp.int32))
    av, ai = lax.fori_loop(0, N_CHUNKS, step, init)

    idx_ref[...] = ai.reshape(1, 1, TOK_BLK)
    part = jnp.sum(av * av).reshape(1, 1)

    @pl.when(i == 0)
    def _init():
        loss_ref[...] = jnp.zeros((1, 1), jnp.float32)

    loss_ref[...] += part


_sc_mesh = plsc.VectorSubcoreMesh(core_axis_name="c", subcore_axis_name="s")


PAD_DIM = 128


@functools.partial(
    pl.kernel,
    mesh=_sc_mesh,
    out_type=jax.ShapeDtypeStruct((N_TOK, PAD_DIM), jnp.float32),
    scratch_types=[
        pltpu.VMEM((ROWS_PER_WORKER,), jnp.int32),
        pltpu.VMEM((ROWS_PER_WORKER, PAD_DIM), jnp.float32),
        pltpu.SemaphoreType.DMA,
    ],
)
def _sc_gather(table_hbm, idx_hbm, out_hbm, idx_v, rows_v, sem):
    wid = lax.axis_index("s") * SC_CORES + lax.axis_index("c")
    base = wid * ROWS_PER_WORKER
    pltpu.sync_copy(idx_hbm.at[pl.ds(base, ROWS_PER_WORKER)], idx_v)
    pltpu.async_copy(table_hbm.at[idx_v], rows_v, sem).wait()
    pltpu.sync_copy(rows_v, out_hbm.at[pl.ds(base, ROWS_PER_WORKER)])


@jax.jit
def kernel(x, embedding):
    flat_x = x.reshape(-1, DIM)
    x_sq = jnp.sum(flat_x * flat_x, axis=1, keepdims=True)
    e_sq = jnp.sum(embedding * embedding, axis=1)[None, :]

    idx3, loss = pl.pallas_call(
        _tc_body,
        grid=(N_BLKS,),
        in_specs=[
            pl.BlockSpec((TOK_BLK, DIM), lambda i: (i, 0)),
            pl.BlockSpec((TOK_BLK, 1), lambda i: (i, 0)),
            pl.BlockSpec((1, N_EMB), lambda i: (0, 0)),
            pl.BlockSpec((N_EMB, DIM), lambda i: (0, 0)),
        ],
        out_specs=[
            pl.BlockSpec((1, 1, TOK_BLK), lambda i: (i, 0, 0)),
            pl.BlockSpec((1, 1), lambda i: (0, 0)),
        ],
        out_shape=[
            jax.ShapeDtypeStruct((N_BLKS, 1, TOK_BLK), jnp.int32),
            jax.ShapeDtypeStruct((1, 1), jnp.float32),
        ],
    )(flat_x, x_sq, e_sq, embedding)

    idx = idx3.reshape(N_TOK)
    emb_pad = jnp.pad(embedding, ((0, 0), (0, PAD_DIM - DIM)))
    q = _sc_gather(emb_pad, idx)[:, :DIM].reshape(x.shape)
    quantized_st = x + (q - x)
    vq_loss = 2.0 * loss[0, 0] / (N_TOK * DIM)
    return quantized_st, vq_loss, idx
